# baseline (speedup 1.0000x reference)
import functools

import jax
import jax.numpy as jnp
import numpy as np
from jax import lax
from jax.experimental import pallas as pl
from jax.experimental.pallas import tpu as pltpu
from jax.experimental.pallas import tpu_sc as plsc

N_NODES = 4096
D_H = 64
K_SIMPLE = 15
K_KNN = 20
NB_PAIR = 5

T_REAL = 15000
T_PAD = 15104
QB = 128


_NSUB = T_PAD // 128


def _bary_knn_body(q_ref, ct_ref, o_ref):
    _BIG = jnp.int32(2**30)
    _INF = jnp.float32(jnp.inf)
    q = q_ref[...]
    acc = None
    for d in range(3):
        diff = q[:, d:d + 1] - ct_ref[d:d + 1, :]
        sq = diff * diff
        acc = sq if acc is None else acc + sq
    d3 = acc.reshape(QB, _NSUB, 128)
    si3 = jax.lax.broadcasted_iota(jnp.int32, (QB, _NSUB, 128), 1)
    lane3 = jax.lax.broadcasted_iota(jnp.int32, (QB, _NSUB, 128), 2)
    flat3 = si3 * 128 + lane3
    lane2 = jax.lax.broadcasted_iota(jnp.int32, (QB, 128), 1)

    def lane_top(dd):
        mv = jnp.min(dd, axis=1)
        sv = jnp.min(jnp.where(dd == mv[:, None, :], si3, _BIG), axis=1)
        return mv, sv

    m1, s1 = lane_top(d3)
    d3a = jnp.where(si3 == s1[:, None, :], _INF, d3)
    m2, s2 = lane_top(d3a)
    d3b = jnp.where(si3 == s2[:, None, :], _INF, d3a)
    m3, s3 = lane_top(d3b)
    i1 = s1 * 128 + lane2
    i2 = s2 * 128 + lane2
    i3 = s3 * 128 + lane2

    cols = []
    for k in range(K_KNN):
        m = jnp.min(m1, axis=1, keepdims=True)
        il = jnp.min(jnp.where(m1 == m, i1, _BIG), axis=1, keepdims=True)
        cols.append(il)
        oh = lane2 == (il % 128)
        ex_any = jnp.any(oh & jnp.isinf(m2) & jnp.isfinite(m1))
        m1 = jnp.where(oh, m2, m1)
        i1 = jnp.where(oh, i2, i1)
        m2 = jnp.where(oh, m3, m2)
        i2 = jnp.where(oh, i3, i2)
        m3 = jnp.where(oh, _INF, m3)
        i3 = jnp.where(oh, _BIG, i3)

        def _refill(ops, _cols=tuple(cols), _oh=oh):
            m1x, i1x = ops
            rem = jnp.zeros((QB, _NSUB, 128), jnp.bool_)
            for o in _cols:
                rem = rem | (flat3 == o[:, :, None])
            dm = jnp.where(rem, _INF, d3)
            nm, nsv = lane_top(dm)
            nf = nsv * 128 + lane2
            mask = _oh & jnp.isinf(m1x)
            return jnp.where(mask, nm, m1x), jnp.where(mask, nf, i1x)

        m1, i1 = jax.lax.cond(ex_any, _refill, lambda ops: ops, (m1, i1))
    o_ref[...] = jnp.concatenate(cols, axis=1)


QT = 64


def _mlp_body(g_ref, bt_ref, wm1_ref, bm1_ref, wm2_ref, o_ref):
    g = g_ref[...][:, :16]
    bt = bt_ref[...]
    r3 = g.reshape(QT, K_KNN, 16) - bt[:, None, :]
    rm = r3.reshape(QT * K_KNN, 16)
    hm = jnp.maximum(
        jnp.dot(rm, wm1_ref[...], preferred_element_type=jnp.float32, precision=jax.lax.Precision.HIGHEST)
        + bm1_ref[...], 0.0)
    hw = hm * rm[:, 9:10]
    hw3 = hw.reshape(QT, K_KNN, 128)
    acc = hw3[:, 0, :]
    for k in range(1, K_KNN):
        acc = acc + hw3[:, k, :]
    pooled = acc / jnp.float32(K_KNN)
    o_ref[...] = jnp.dot(pooled, wm2_ref[...], preferred_element_type=jnp.float32, precision=jax.lax.Precision.HIGHEST)


def _mlp_scores(rowsg, bary_tile16, Wm1, bm1, Wm2, bm2):
    wm1p = jnp.pad(Wm1, ((0, 7), (0, 0)))
    fs = pl.pallas_call(
        _mlp_body,
        grid=(T_PAD // QT,),
        in_specs=[
            pl.BlockSpec((QT * K_KNN, 128), lambda i: (i, 0)),
            pl.BlockSpec((QT, 16), lambda i: (i, 0)),
            pl.BlockSpec((16, 128), lambda i: (0, 0)),
            pl.BlockSpec((1, 128), lambda i: (0, 0)),
            pl.BlockSpec((128, 1), lambda i: (0, 0)),
        ],
        out_specs=pl.BlockSpec((QT, 1), lambda i: (i, 0)),
        out_shape=jax.ShapeDtypeStruct((T_PAD, 1), jnp.float32),
    )(rowsg[:T_PAD * K_KNN], bary_tile16, wm1p, bm1.reshape(1, 128), Wm2)
    return fs + bm2


RB = 128


def _rank_body(s_ref, st_ref, o_ref):
    s = s_ref[...]
    st = st_ref[...][0:1, :]
    j = jax.lax.broadcasted_iota(jnp.int32, (RB, T_PAD), 1)
    i = (jax.lax.broadcasted_iota(jnp.int32, (RB, 1), 0)
         + pl.program_id(0) * RB)
    gt = (st > s) | ((st == s) & (j < i))
    o_ref[...] = jnp.sum(gt.astype(jnp.int32), axis=1, keepdims=True)


OB = 256


def _build_body(rk_ref, tri_ref, o_ref):
    rk = rk_ref[...][0:1, :]
    r = (jax.lax.broadcasted_iota(jnp.int32, (OB, 1), 0)
         + pl.program_id(0) * OB)
    M = (rk == r).astype(jnp.float32)
    o_ref[...] = jnp.dot(M, tri_ref[...], preferred_element_type=jnp.float32, precision=jax.lax.Precision.HIGHEST)


def _select_top500(fs_pad, tri16):
    rank = pl.pallas_call(
        _rank_body,
        grid=(T_PAD // RB,),
        in_specs=[
            pl.BlockSpec((RB, 1), lambda i: (i, 0)),
            pl.BlockSpec((8, T_PAD), lambda i: (0, 0)),
        ],
        out_specs=pl.BlockSpec((RB, 1), lambda i: (i, 0)),
        out_shape=jax.ShapeDtypeStruct((T_PAD, 1), jnp.int32),
    )(fs_pad, jnp.broadcast_to(fs_pad.T, (8, T_PAD)))
    out = pl.pallas_call(
        _build_body,
        grid=(2,),
        in_specs=[
            pl.BlockSpec((8, T_PAD), lambda i: (0, 0)),
            pl.BlockSpec((T_PAD, 16), lambda i: (0, 0)),
        ],
        out_specs=pl.BlockSpec((OB, 16), lambda i: (i, 0)),
        out_shape=jax.ShapeDtypeStruct((2 * OB, 16), jnp.float32),
    )(jnp.broadcast_to(rank.T, (8, T_PAD)), tri16)
    return out


_NW = 32
_CH = 384


def _sc_gather_rows(table, idx, n_chunks):
    mesh = plsc.VectorSubcoreMesh(core_axis_name="c", subcore_axis_name="s")
    b_total = idx.shape[0]

    @functools.partial(
        pl.kernel,
        mesh=mesh,
        out_type=jax.ShapeDtypeStruct((b_total, 128), jnp.float32),
        scratch_types=[
            pltpu.VMEM((_CH,), jnp.int32),
            pltpu.VMEM((_CH,), jnp.int32),
            pltpu.VMEM((_CH, 128), jnp.float32),
            pltpu.VMEM((_CH, 128), jnp.float32),
            pltpu.SemaphoreType.DMA,
            pltpu.SemaphoreType.DMA,
        ],
    )
    def gk(table_hbm, idx_hbm, out_hbm, idx0, idx1, rows0, rows1, sem0, sem1):
        wid = lax.axis_index("s") * 2 + lax.axis_index("c")
        w0 = wid * (n_chunks * _CH)
        idx_v = (idx0, idx1)
        rows_v = (rows0, rows1)
        sems = (sem0, sem1)
        pltpu.sync_copy(idx_hbm.at[pl.ds(w0, _CH)], idx0)
        cp = pltpu.async_copy(table_hbm.at[idx0], rows0, sem0)
        for c in range(n_chunks):
            p, q = c % 2, (c + 1) % 2
            if c + 1 < n_chunks:
                pltpu.sync_copy(idx_hbm.at[pl.ds(w0 + (c + 1) * _CH, _CH)], idx_v[q])
                ncp = pltpu.async_copy(table_hbm.at[idx_v[q]], rows_v[q], sems[q])
            cp.wait()
            pltpu.sync_copy(rows_v[p], out_hbm.at[pl.ds(w0 + c * _CH, _CH)])
            if c + 1 < n_chunks:
                cp = ncp

    return gk(table, idx)


def _bary_knn(bary):
    baryp = jnp.concatenate(
        [bary, jnp.full((T_PAD - T_REAL, 3), 1e20, jnp.float32)], axis=0)
    baryp = jnp.pad(baryp, ((0, 0), (0, 5)))
    baryT = baryp.T
    nbr = pl.pallas_call(
        _bary_knn_body,
        grid=(T_PAD // QB,),
        in_specs=[
            pl.BlockSpec((QB, 8), lambda i: (i, 0)),
            pl.BlockSpec((8, T_PAD), lambda i: (0, 0)),
        ],
        out_specs=pl.BlockSpec((QB, K_KNN), lambda i: (i, 0)),
        out_shape=jax.ShapeDtypeStruct((T_PAD, K_KNN), jnp.int32),
        compiler_params=pltpu.CompilerParams(dimension_semantics=("parallel",)),
    )(baryp, baryT)
    return nbr[:T_REAL]


def kernel(user_number_triangles, graph_nodes, graph_adjacency_matrix, W1, b1, W2, Wdev, Wq, Wk, Wm1, bm1, Wm2, bm2):
    A = graph_adjacency_matrix
    A_norm = A / (jnp.sum(A, axis=1, keepdims=True) + 1e-6)
    h = jax.nn.relu(A_norm @ (graph_nodes @ W1) + b1)
    inclusion_score = (A_norm @ (h @ W2))[:, 0]
    N_TRI = 500
    target_p = min(graph_nodes.shape[0], N_TRI * 3)
    u = jax.random.uniform(jax.random.key(42), inclusion_score.shape, dtype=jnp.float32)
    g = -jnp.log(-jnp.log(u + 1e-20) + 1e-20)
    _, sel = jax.lax.top_k(jax.lax.stop_gradient(inclusion_score) + g, target_p)
    x = graph_nodes[sel]
    x_sg = x
    d2 = jnp.sum((x_sg[:, None, :] - x_sg[None, :, :]) ** 2, axis=-1)
    _, nn_idx = jax.lax.top_k(-d2, K_SIMPLE + 1)
    knn = nn_idx[:, 1:]
    xdiff = x[knn] - x[:, None, :]
    edge_feat = jax.nn.relu(xdiff @ Wdev)
    f = jnp.mean(edge_feat, axis=1)
    q = f @ Wq
    kk = f @ Wk
    att = jnp.einsum('pd,pkd->pk', q, kk[knn]) / jnp.sqrt(float(D_H))
    S = jax.nn.sigmoid(att)
    P = x.shape[0]
    rows = jnp.broadcast_to(jnp.arange(P)[:, None], knn.shape)
    A_s = jnp.zeros((P, P), dtype=jnp.float32).at[rows, knn].max(S)
    A_s = jnp.maximum(A_s, A_s.T)
    pa, pb = np.triu_indices(NB_PAIR, 1)
    anchor = jnp.broadcast_to(jnp.arange(P)[:, None], (P, pa.shape[0]))
    tri_ids = jnp.stack([anchor, knn[:, pa], knn[:, pb]], axis=-1).reshape(-1, 3)
    triangles = x[tri_ids]
    i0, i1, i2 = tri_ids[:, 0], tri_ids[:, 1], tri_ids[:, 2]
    p_init = A_s[i0, i1] * A_s[i1, i2] * A_s[i0, i2]
    bary = jnp.mean(triangles, axis=1)
    bary_sg = bary
    T = bary.shape[0]
    CH = 500

    indices_neigh_tri = _bary_knn(bary_sg)
    tp = jnp.concatenate(
        [triangles.reshape(T, 9), p_init[:, None], jnp.zeros((T, 118), jnp.float32)],
        axis=1)
    n_chunks = -(-(T_PAD * K_KNN) // (_NW * _CH))
    b_pad = _NW * n_chunks * _CH
    idx_flat = jnp.pad(
        jnp.pad(indices_neigh_tri, ((0, T_PAD - T), (0, 0))).reshape(-1),
        (0, b_pad - T_PAD * K_KNN))
    rowsg = _sc_gather_rows(tp, idx_flat.astype(jnp.int32), n_chunks)
    g16 = rowsg[:T * K_KNN].reshape(T, K_KNN, 128)
    r_matrix = g16[:, :, :9] - jnp.tile(bary, (1, 3))[:, None, :]
    w = g16[:, :, 9:10]
    hm = jax.nn.relu(r_matrix @ Wm1 + bm1)
    pooled = jnp.mean(hm * w, axis=1)
    final_scores = (pooled @ Wm2 + bm2)[:, 0]
    final_scores = final_scores + 0.0 * jnp.asarray(user_number_triangles, dtype=jnp.float32)
    fs_pad = jnp.pad(final_scores[:, None], ((0, T_PAD - T), (0, 0)),
                     constant_values=-jnp.inf)
    tri16 = jnp.pad(triangles.reshape(T, 9), ((0, T_PAD - T), (0, 7)))
    out = _select_top500(fs_pad, tri16)
    return out[:N_TRI, :9].reshape(N_TRI, 3, 3)


# speedup vs baseline: 3.3473x; 3.3473x over previous
import functools

import jax
import jax.numpy as jnp
import numpy as np
from jax import lax
from jax.experimental import pallas as pl
from jax.experimental.pallas import tpu as pltpu
from jax.experimental.pallas import tpu_sc as plsc

N_NODES = 4096
D_H = 64
K_SIMPLE = 15
K_KNN = 20
NB_PAIR = 5

T_REAL = 15000
T_PAD = 15104
QB = 128


_NSUB = T_PAD // 128


def _bary_knn_body(q_ref, ct_ref, o_ref):
    _BIG = jnp.int32(2**30)
    _INF = jnp.float32(jnp.inf)
    q = q_ref[...]
    acc = None
    for d in range(3):
        diff = q[:, d:d + 1] - ct_ref[d:d + 1, :]
        sq = diff * diff
        acc = sq if acc is None else acc + sq
    iota = jax.lax.broadcasted_iota(jnp.int32, (QB, T_PAD), 1)
    d2 = acc
    cols = []
    for k in range(K_KNN):
        m = jnp.min(d2, axis=1, keepdims=True)
        im = jnp.min(jnp.where(d2 == m, iota, _BIG), axis=1, keepdims=True)
        cols.append(im)
        d2 = jnp.where(iota == im, _INF, d2)
    o_ref[...] = jnp.concatenate(cols, axis=1)


QT = 64


def _mlp_body(g_ref, bt_ref, wm1_ref, bm1_ref, wm2_ref, o_ref):
    g = g_ref[...][:, :16]
    bt = bt_ref[...]
    r3 = g.reshape(QT, K_KNN, 16) - bt[:, None, :]
    rm = r3.reshape(QT * K_KNN, 16)
    hm = jnp.maximum(
        jnp.dot(rm, wm1_ref[...], preferred_element_type=jnp.float32, precision=jax.lax.Precision.HIGHEST)
        + bm1_ref[...], 0.0)
    hw = hm * rm[:, 9:10]
    hw3 = hw.reshape(QT, K_KNN, 128)
    acc = hw3[:, 0, :]
    for k in range(1, K_KNN):
        acc = acc + hw3[:, k, :]
    pooled = acc / jnp.float32(K_KNN)
    o_ref[...] = jnp.dot(pooled, wm2_ref[...], preferred_element_type=jnp.float32, precision=jax.lax.Precision.HIGHEST)


def _mlp_scores(rowsg, bary_tile16, Wm1, bm1, Wm2, bm2):
    wm1p = jnp.pad(Wm1, ((0, 7), (0, 0)))
    fs = pl.pallas_call(
        _mlp_body,
        grid=(T_PAD // QT,),
        in_specs=[
            pl.BlockSpec((QT * K_KNN, 128), lambda i: (i, 0)),
            pl.BlockSpec((QT, 16), lambda i: (i, 0)),
            pl.BlockSpec((16, 128), lambda i: (0, 0)),
            pl.BlockSpec((1, 128), lambda i: (0, 0)),
            pl.BlockSpec((128, 1), lambda i: (0, 0)),
        ],
        out_specs=pl.BlockSpec((QT, 1), lambda i: (i, 0)),
        out_shape=jax.ShapeDtypeStruct((T_PAD, 1), jnp.float32),
    )(rowsg[:T_PAD * K_KNN], bary_tile16, wm1p, bm1.reshape(1, 128), Wm2)
    return fs + bm2


RB = 128


def _rank_body(s_ref, st_ref, o_ref):
    s = s_ref[...]
    st = st_ref[...][0:1, :]
    j = jax.lax.broadcasted_iota(jnp.int32, (RB, T_PAD), 1)
    i = (jax.lax.broadcasted_iota(jnp.int32, (RB, 1), 0)
         + pl.program_id(0) * RB)
    gt = (st > s) | ((st == s) & (j < i))
    o_ref[...] = jnp.sum(gt.astype(jnp.int32), axis=1, keepdims=True)


OB = 256


def _build_body(rk_ref, tri_ref, o_ref):
    rk = rk_ref[...][0:1, :]
    r = (jax.lax.broadcasted_iota(jnp.int32, (OB, 1), 0)
         + pl.program_id(0) * OB)
    M = (rk == r).astype(jnp.float32)
    o_ref[...] = jnp.dot(M, tri_ref[...], preferred_element_type=jnp.float32, precision=jax.lax.Precision.HIGHEST)


def _select_top500(fs_pad, tri16):
    rank = pl.pallas_call(
        _rank_body,
        grid=(T_PAD // RB,),
        in_specs=[
            pl.BlockSpec((RB, 1), lambda i: (i, 0)),
            pl.BlockSpec((8, T_PAD), lambda i: (0, 0)),
        ],
        out_specs=pl.BlockSpec((RB, 1), lambda i: (i, 0)),
        out_shape=jax.ShapeDtypeStruct((T_PAD, 1), jnp.int32),
    )(fs_pad, jnp.broadcast_to(fs_pad.T, (8, T_PAD)))
    out = pl.pallas_call(
        _build_body,
        grid=(2,),
        in_specs=[
            pl.BlockSpec((8, T_PAD), lambda i: (0, 0)),
            pl.BlockSpec((T_PAD, 16), lambda i: (0, 0)),
        ],
        out_specs=pl.BlockSpec((OB, 16), lambda i: (i, 0)),
        out_shape=jax.ShapeDtypeStruct((2 * OB, 16), jnp.float32),
    )(jnp.broadcast_to(rank.T, (8, T_PAD)), tri16)
    return out


_NW = 32
_CH = 384


def _sc_gather_rows(table, idx, n_chunks):
    mesh = plsc.VectorSubcoreMesh(core_axis_name="c", subcore_axis_name="s")
    b_total = idx.shape[0]

    @functools.partial(
        pl.kernel,
        mesh=mesh,
        out_type=jax.ShapeDtypeStruct((b_total, 128), jnp.float32),
        scratch_types=[
            pltpu.VMEM((_CH,), jnp.int32),
            pltpu.VMEM((_CH,), jnp.int32),
            pltpu.VMEM((_CH, 128), jnp.float32),
            pltpu.VMEM((_CH, 128), jnp.float32),
            pltpu.SemaphoreType.DMA,
            pltpu.SemaphoreType.DMA,
        ],
    )
    def gk(table_hbm, idx_hbm, out_hbm, idx0, idx1, rows0, rows1, sem0, sem1):
        wid = lax.axis_index("s") * 2 + lax.axis_index("c")
        w0 = wid * (n_chunks * _CH)
        idx_v = (idx0, idx1)
        rows_v = (rows0, rows1)
        sems = (sem0, sem1)
        pltpu.sync_copy(idx_hbm.at[pl.ds(w0, _CH)], idx0)
        cp = pltpu.async_copy(table_hbm.at[idx0], rows0, sem0)
        for c in range(n_chunks):
            p, q = c % 2, (c + 1) % 2
            if c + 1 < n_chunks:
                pltpu.sync_copy(idx_hbm.at[pl.ds(w0 + (c + 1) * _CH, _CH)], idx_v[q])
                ncp = pltpu.async_copy(table_hbm.at[idx_v[q]], rows_v[q], sems[q])
            cp.wait()
            pltpu.sync_copy(rows_v[p], out_hbm.at[pl.ds(w0 + c * _CH, _CH)])
            if c + 1 < n_chunks:
                cp = ncp

    return gk(table, idx)


def _bary_knn(bary):
    baryp = jnp.concatenate(
        [bary, jnp.full((T_PAD - T_REAL, 3), 1e20, jnp.float32)], axis=0)
    baryp = jnp.pad(baryp, ((0, 0), (0, 5)))
    baryT = baryp.T
    nbr = pl.pallas_call(
        _bary_knn_body,
        grid=(T_PAD // QB,),
        in_specs=[
            pl.BlockSpec((QB, 8), lambda i: (i, 0)),
            pl.BlockSpec((8, T_PAD), lambda i: (0, 0)),
        ],
        out_specs=pl.BlockSpec((QB, K_KNN), lambda i: (i, 0)),
        out_shape=jax.ShapeDtypeStruct((T_PAD, K_KNN), jnp.int32),
        compiler_params=pltpu.CompilerParams(dimension_semantics=("parallel",)),
    )(baryp, baryT)
    return nbr[:T_REAL]


def kernel(user_number_triangles, graph_nodes, graph_adjacency_matrix, W1, b1, W2, Wdev, Wq, Wk, Wm1, bm1, Wm2, bm2):
    A = graph_adjacency_matrix
    A_norm = A / (jnp.sum(A, axis=1, keepdims=True) + 1e-6)
    h = jax.nn.relu(A_norm @ (graph_nodes @ W1) + b1)
    inclusion_score = (A_norm @ (h @ W2))[:, 0]
    N_TRI = 500
    target_p = min(graph_nodes.shape[0], N_TRI * 3)
    u = jax.random.uniform(jax.random.key(42), inclusion_score.shape, dtype=jnp.float32)
    g = -jnp.log(-jnp.log(u + 1e-20) + 1e-20)
    _, sel = jax.lax.top_k(jax.lax.stop_gradient(inclusion_score) + g, target_p)
    x = graph_nodes[sel]
    x_sg = x
    d2 = jnp.sum((x_sg[:, None, :] - x_sg[None, :, :]) ** 2, axis=-1)
    _, nn_idx = jax.lax.top_k(-d2, K_SIMPLE + 1)
    knn = nn_idx[:, 1:]
    xdiff = x[knn] - x[:, None, :]
    edge_feat = jax.nn.relu(xdiff @ Wdev)
    f = jnp.mean(edge_feat, axis=1)
    q = f @ Wq
    kk = f @ Wk
    att = jnp.einsum('pd,pkd->pk', q, kk[knn]) / jnp.sqrt(float(D_H))
    S = jax.nn.sigmoid(att)
    P = x.shape[0]
    rows = jnp.broadcast_to(jnp.arange(P)[:, None], knn.shape)
    A_s = jnp.zeros((P, P), dtype=jnp.float32).at[rows, knn].max(S)
    A_s = jnp.maximum(A_s, A_s.T)
    pa, pb = np.triu_indices(NB_PAIR, 1)
    anchor = jnp.broadcast_to(jnp.arange(P)[:, None], (P, pa.shape[0]))
    tri_ids = jnp.stack([anchor, knn[:, pa], knn[:, pb]], axis=-1).reshape(-1, 3)
    triangles = x[tri_ids]
    i0, i1, i2 = tri_ids[:, 0], tri_ids[:, 1], tri_ids[:, 2]
    p_init = A_s[i0, i1] * A_s[i1, i2] * A_s[i0, i2]
    bary = jnp.mean(triangles, axis=1)
    bary_sg = bary
    T = bary.shape[0]
    CH = 500

    indices_neigh_tri = _bary_knn(bary_sg)
    tp = jnp.concatenate(
        [triangles.reshape(T, 9), p_init[:, None], jnp.zeros((T, 118), jnp.float32)],
        axis=1)
    n_chunks = -(-(T_PAD * K_KNN) // (_NW * _CH))
    b_pad = _NW * n_chunks * _CH
    idx_flat = jnp.pad(
        jnp.pad(indices_neigh_tri, ((0, T_PAD - T), (0, 0))).reshape(-1),
        (0, b_pad - T_PAD * K_KNN))
    rowsg = _sc_gather_rows(tp, idx_flat.astype(jnp.int32), n_chunks)
    g16 = rowsg[:T * K_KNN].reshape(T, K_KNN, 128)
    r_matrix = g16[:, :, :9] - jnp.tile(bary, (1, 3))[:, None, :]
    w = g16[:, :, 9:10]
    hm = jax.nn.relu(r_matrix @ Wm1 + bm1)
    pooled = jnp.mean(hm * w, axis=1)
    final_scores = (pooled @ Wm2 + bm2)[:, 0]
    final_scores = final_scores + 0.0 * jnp.asarray(user_number_triangles, dtype=jnp.float32)
    fs_pad = jnp.pad(final_scores[:, None], ((0, T_PAD - T), (0, 0)),
                     constant_values=-jnp.inf)
    tri16 = jnp.pad(triangles.reshape(T, 9), ((0, T_PAD - T), (0, 7)))
    out = _select_top500(fs_pad, tri16)
    return out[:N_TRI, :9].reshape(N_TRI, 3, 3)


# speedup vs baseline: 3.6593x; 1.0932x over previous
import functools

import jax
import jax.numpy as jnp
import numpy as np
from jax import lax
from jax.experimental import pallas as pl
from jax.experimental.pallas import tpu as pltpu
from jax.experimental.pallas import tpu_sc as plsc

N_NODES = 4096
D_H = 64
K_SIMPLE = 15
K_KNN = 20
NB_PAIR = 5

T_REAL = 15000
T_PAD = 15104
QB = 256


_NSUB = T_PAD // 128


def _bary_knn_body(q_ref, ct_ref, o_ref):
    _BIG = jnp.int32(2**30)
    _INF = jnp.float32(jnp.inf)
    q = q_ref[...]
    acc = None
    for d in range(3):
        diff = q[:, d:d + 1] - ct_ref[d:d + 1, :]
        sq = diff * diff
        acc = sq if acc is None else acc + sq
    iota = jax.lax.broadcasted_iota(jnp.int32, (QB, T_PAD), 1)
    d2 = acc
    cols = []
    for k in range(K_KNN):
        m = jnp.min(d2, axis=1, keepdims=True)
        im = jnp.min(jnp.where(d2 == m, iota, _BIG), axis=1, keepdims=True)
        cols.append(im)
        d2 = jnp.where(iota == im, _INF, d2)
    o_ref[...] = jnp.concatenate(cols, axis=1)


QT = 64


def _mlp_body(g_ref, bt_ref, wm1_ref, bm1_ref, wm2_ref, o_ref):
    g = g_ref[...][:, :16]
    bt = bt_ref[...]
    r3 = g.reshape(QT, K_KNN, 16) - bt[:, None, :]
    rm = r3.reshape(QT * K_KNN, 16)
    hm = jnp.maximum(
        jnp.dot(rm, wm1_ref[...], preferred_element_type=jnp.float32, precision=jax.lax.Precision.HIGHEST)
        + bm1_ref[...], 0.0)
    hw = hm * rm[:, 9:10]
    hw3 = hw.reshape(QT, K_KNN, 128)
    acc = hw3[:, 0, :]
    for k in range(1, K_KNN):
        acc = acc + hw3[:, k, :]
    pooled = acc / jnp.float32(K_KNN)
    o_ref[...] = jnp.dot(pooled, wm2_ref[...], preferred_element_type=jnp.float32, precision=jax.lax.Precision.HIGHEST)


def _mlp_scores(rowsg, bary_tile16, Wm1, bm1, Wm2, bm2):
    wm1p = jnp.pad(Wm1, ((0, 7), (0, 0)))
    fs = pl.pallas_call(
        _mlp_body,
        grid=(T_PAD // QT,),
        in_specs=[
            pl.BlockSpec((QT * K_KNN, 128), lambda i: (i, 0)),
            pl.BlockSpec((QT, 16), lambda i: (i, 0)),
            pl.BlockSpec((16, 128), lambda i: (0, 0)),
            pl.BlockSpec((1, 128), lambda i: (0, 0)),
            pl.BlockSpec((128, 1), lambda i: (0, 0)),
        ],
        out_specs=pl.BlockSpec((QT, 1), lambda i: (i, 0)),
        out_shape=jax.ShapeDtypeStruct((T_PAD, 1), jnp.float32),
    )(rowsg[:T_PAD * K_KNN], bary_tile16, wm1p, bm1.reshape(1, 128), Wm2)
    return fs + bm2


RB = 128


def _rank_body(s_ref, st_ref, o_ref):
    s = s_ref[...]
    st = st_ref[...][0:1, :]
    j = jax.lax.broadcasted_iota(jnp.int32, (RB, T_PAD), 1)
    i = (jax.lax.broadcasted_iota(jnp.int32, (RB, 1), 0)
         + pl.program_id(0) * RB)
    gt = (st > s) | ((st == s) & (j < i))
    o_ref[...] = jnp.sum(gt.astype(jnp.int32), axis=1, keepdims=True)


OB = 256


def _build_body(rk_ref, tri_ref, o_ref):
    rk = rk_ref[...][0:1, :]
    r = (jax.lax.broadcasted_iota(jnp.int32, (OB, 1), 0)
         + pl.program_id(0) * OB)
    M = (rk == r).astype(jnp.float32)
    o_ref[...] = jnp.dot(M, tri_ref[...], preferred_element_type=jnp.float32, precision=jax.lax.Precision.HIGHEST)


def _select_top500(fs_pad, tri16):
    rank = pl.pallas_call(
        _rank_body,
        grid=(T_PAD // RB,),
        in_specs=[
            pl.BlockSpec((RB, 1), lambda i: (i, 0)),
            pl.BlockSpec((8, T_PAD), lambda i: (0, 0)),
        ],
        out_specs=pl.BlockSpec((RB, 1), lambda i: (i, 0)),
        out_shape=jax.ShapeDtypeStruct((T_PAD, 1), jnp.int32),
    )(fs_pad, jnp.broadcast_to(fs_pad.T, (8, T_PAD)))
    out = pl.pallas_call(
        _build_body,
        grid=(2,),
        in_specs=[
            pl.BlockSpec((8, T_PAD), lambda i: (0, 0)),
            pl.BlockSpec((T_PAD, 16), lambda i: (0, 0)),
        ],
        out_specs=pl.BlockSpec((OB, 16), lambda i: (i, 0)),
        out_shape=jax.ShapeDtypeStruct((2 * OB, 16), jnp.float32),
    )(jnp.broadcast_to(rank.T, (8, T_PAD)), tri16)
    return out


_NW = 32
_CH = 384


def _sc_gather_rows(table, idx, n_chunks):
    mesh = plsc.VectorSubcoreMesh(core_axis_name="c", subcore_axis_name="s")
    b_total = idx.shape[0]

    @functools.partial(
        pl.kernel,
        mesh=mesh,
        out_type=jax.ShapeDtypeStruct((b_total, 128), jnp.float32),
        scratch_types=[
            pltpu.VMEM((_CH,), jnp.int32),
            pltpu.VMEM((_CH,), jnp.int32),
            pltpu.VMEM((_CH, 128), jnp.float32),
            pltpu.VMEM((_CH, 128), jnp.float32),
            pltpu.SemaphoreType.DMA,
            pltpu.SemaphoreType.DMA,
        ],
    )
    def gk(table_hbm, idx_hbm, out_hbm, idx0, idx1, rows0, rows1, sem0, sem1):
        wid = lax.axis_index("s") * 2 + lax.axis_index("c")
        w0 = wid * (n_chunks * _CH)
        idx_v = (idx0, idx1)
        rows_v = (rows0, rows1)
        sems = (sem0, sem1)
        pltpu.sync_copy(idx_hbm.at[pl.ds(w0, _CH)], idx0)
        cp = pltpu.async_copy(table_hbm.at[idx0], rows0, sem0)
        for c in range(n_chunks):
            p, q = c % 2, (c + 1) % 2
            if c + 1 < n_chunks:
                pltpu.sync_copy(idx_hbm.at[pl.ds(w0 + (c + 1) * _CH, _CH)], idx_v[q])
                ncp = pltpu.async_copy(table_hbm.at[idx_v[q]], rows_v[q], sems[q])
            cp.wait()
            pltpu.sync_copy(rows_v[p], out_hbm.at[pl.ds(w0 + c * _CH, _CH)])
            if c + 1 < n_chunks:
                cp = ncp

    return gk(table, idx)


def _bary_knn(bary):
    baryp = jnp.concatenate(
        [bary, jnp.full((T_PAD - T_REAL, 3), 1e20, jnp.float32)], axis=0)
    baryp = jnp.pad(baryp, ((0, 0), (0, 5)))
    baryT = baryp.T
    nbr = pl.pallas_call(
        _bary_knn_body,
        grid=(T_PAD // QB,),
        in_specs=[
            pl.BlockSpec((QB, 8), lambda i: (i, 0)),
            pl.BlockSpec((8, T_PAD), lambda i: (0, 0)),
        ],
        out_specs=pl.BlockSpec((QB, K_KNN), lambda i: (i, 0)),
        out_shape=jax.ShapeDtypeStruct((T_PAD, K_KNN), jnp.int32),
        compiler_params=pltpu.CompilerParams(dimension_semantics=("parallel",)),
    )(baryp, baryT)
    return nbr[:T_REAL]


def kernel(user_number_triangles, graph_nodes, graph_adjacency_matrix, W1, b1, W2, Wdev, Wq, Wk, Wm1, bm1, Wm2, bm2):
    A = graph_adjacency_matrix
    A_norm = A / (jnp.sum(A, axis=1, keepdims=True) + 1e-6)
    h = jax.nn.relu(A_norm @ (graph_nodes @ W1) + b1)
    inclusion_score = (A_norm @ (h @ W2))[:, 0]
    N_TRI = 500
    target_p = min(graph_nodes.shape[0], N_TRI * 3)
    u = jax.random.uniform(jax.random.key(42), inclusion_score.shape, dtype=jnp.float32)
    g = -jnp.log(-jnp.log(u + 1e-20) + 1e-20)
    _, sel = jax.lax.top_k(jax.lax.stop_gradient(inclusion_score) + g, target_p)
    x = graph_nodes[sel]
    x_sg = x
    d2 = jnp.sum((x_sg[:, None, :] - x_sg[None, :, :]) ** 2, axis=-1)
    _, nn_idx = jax.lax.top_k(-d2, K_SIMPLE + 1)
    knn = nn_idx[:, 1:]
    xdiff = x[knn] - x[:, None, :]
    edge_feat = jax.nn.relu(xdiff @ Wdev)
    f = jnp.mean(edge_feat, axis=1)
    q = f @ Wq
    kk = f @ Wk
    att = jnp.einsum('pd,pkd->pk', q, kk[knn]) / jnp.sqrt(float(D_H))
    S = jax.nn.sigmoid(att)
    P = x.shape[0]
    rows = jnp.broadcast_to(jnp.arange(P)[:, None], knn.shape)
    A_s = jnp.zeros((P, P), dtype=jnp.float32).at[rows, knn].max(S)
    A_s = jnp.maximum(A_s, A_s.T)
    pa, pb = np.triu_indices(NB_PAIR, 1)
    anchor = jnp.broadcast_to(jnp.arange(P)[:, None], (P, pa.shape[0]))
    tri_ids = jnp.stack([anchor, knn[:, pa], knn[:, pb]], axis=-1).reshape(-1, 3)
    triangles = x[tri_ids]
    i0, i1, i2 = tri_ids[:, 0], tri_ids[:, 1], tri_ids[:, 2]
    p_init = A_s[i0, i1] * A_s[i1, i2] * A_s[i0, i2]
    bary = jnp.mean(triangles, axis=1)
    bary_sg = bary
    T = bary.shape[0]
    CH = 500

    indices_neigh_tri = _bary_knn(bary_sg)
    tp = jnp.concatenate(
        [triangles.reshape(T, 9), p_init[:, None], jnp.zeros((T, 118), jnp.float32)],
        axis=1)
    n_chunks = -(-(T_PAD * K_KNN) // (_NW * _CH))
    b_pad = _NW * n_chunks * _CH
    idx_flat = jnp.pad(
        jnp.pad(indices_neigh_tri, ((0, T_PAD - T), (0, 0))).reshape(-1),
        (0, b_pad - T_PAD * K_KNN))
    rowsg = _sc_gather_rows(tp, idx_flat.astype(jnp.int32), n_chunks)
    g16 = rowsg[:T * K_KNN].reshape(T, K_KNN, 128)
    r_matrix = g16[:, :, :9] - jnp.tile(bary, (1, 3))[:, None, :]
    w = g16[:, :, 9:10]
    hm = jax.nn.relu(r_matrix @ Wm1 + bm1)
    pooled = jnp.mean(hm * w, axis=1)
    final_scores = (pooled @ Wm2 + bm2)[:, 0]
    final_scores = final_scores + 0.0 * jnp.asarray(user_number_triangles, dtype=jnp.float32)
    fs_pad = jnp.pad(final_scores[:, None], ((0, T_PAD - T), (0, 0)),
                     constant_values=-jnp.inf)
    tri16 = jnp.pad(triangles.reshape(T, 9), ((0, T_PAD - T), (0, 7)))
    out = _select_top500(fs_pad, tri16)
    return out[:N_TRI, :9].reshape(N_TRI, 3, 3)


# speedup vs baseline: 3.6599x; 1.0001x over previous
import functools

import jax
import jax.numpy as jnp
import numpy as np
from jax import lax
from jax.experimental import pallas as pl
from jax.experimental.pallas import tpu as pltpu
from jax.experimental.pallas import tpu_sc as plsc

N_NODES = 4096
D_H = 64
K_SIMPLE = 15
K_KNN = 20
NB_PAIR = 5

T_REAL = 15000
T_PAD = 15104
QB = 256


def _bary_knn_body(q_ref, ct_ref, o_ref):
    _BIG = jnp.int32(2**30)
    _INF = jnp.float32(jnp.inf)
    q = q_ref[...]
    acc = None
    for d in range(3):
        diff = q[:, d:d + 1] - ct_ref[d:d + 1, :]
        sq = diff * diff
        acc = sq if acc is None else acc + sq
    iota = jax.lax.broadcasted_iota(jnp.int32, (QB, T_PAD), 1)
    d2 = acc
    cols = []
    for k in range(K_KNN):
        m = jnp.min(d2, axis=1, keepdims=True)
        im = jnp.min(jnp.where(d2 == m, iota, _BIG), axis=1, keepdims=True)
        cols.append(im)
        d2 = jnp.where(iota == im, _INF, d2)
    o_ref[...] = jnp.concatenate(cols, axis=1)


RB = 128


def _rank_body(s_ref, st_ref, o_ref):
    s = s_ref[...]
    st = st_ref[...][0:1, :]
    j = jax.lax.broadcasted_iota(jnp.int32, (RB, T_PAD), 1)
    i = (jax.lax.broadcasted_iota(jnp.int32, (RB, 1), 0)
         + pl.program_id(0) * RB)
    gt = (st > s) | ((st == s) & (j < i))
    o_ref[...] = jnp.sum(gt.astype(jnp.int32), axis=1, keepdims=True)


OB = 256


def _build_body(rk_ref, tri_ref, o_ref):
    rk = rk_ref[...][0:1, :]
    r = (jax.lax.broadcasted_iota(jnp.int32, (OB, 1), 0)
         + pl.program_id(0) * OB)
    M = (rk == r).astype(jnp.float32)
    o_ref[...] = jnp.dot(M, tri_ref[...], preferred_element_type=jnp.float32, precision=jax.lax.Precision.HIGHEST)


def _select_top500(fs_pad, tri16):
    rank = pl.pallas_call(
        _rank_body,
        grid=(T_PAD // RB,),
        in_specs=[
            pl.BlockSpec((RB, 1), lambda i: (i, 0)),
            pl.BlockSpec((8, T_PAD), lambda i: (0, 0)),
        ],
        out_specs=pl.BlockSpec((RB, 1), lambda i: (i, 0)),
        out_shape=jax.ShapeDtypeStruct((T_PAD, 1), jnp.int32),
    )(fs_pad, jnp.broadcast_to(fs_pad.T, (8, T_PAD)))
    out = pl.pallas_call(
        _build_body,
        grid=(2,),
        in_specs=[
            pl.BlockSpec((8, T_PAD), lambda i: (0, 0)),
            pl.BlockSpec((T_PAD, 16), lambda i: (0, 0)),
        ],
        out_specs=pl.BlockSpec((OB, 16), lambda i: (i, 0)),
        out_shape=jax.ShapeDtypeStruct((2 * OB, 16), jnp.float32),
    )(jnp.broadcast_to(rank.T, (8, T_PAD)), tri16)
    return out


_NW = 32
_CH = 384


def _sc_gather_rows(table, idx, n_chunks):
    mesh = plsc.VectorSubcoreMesh(core_axis_name="c", subcore_axis_name="s")
    b_total = idx.shape[0]

    @functools.partial(
        pl.kernel,
        mesh=mesh,
        out_type=jax.ShapeDtypeStruct((b_total, 128), jnp.float32),
        scratch_types=[
            pltpu.VMEM((_CH,), jnp.int32),
            pltpu.VMEM((_CH,), jnp.int32),
            pltpu.VMEM((_CH, 128), jnp.float32),
            pltpu.VMEM((_CH, 128), jnp.float32),
            pltpu.SemaphoreType.DMA,
            pltpu.SemaphoreType.DMA,
        ],
    )
    def gk(table_hbm, idx_hbm, out_hbm, idx0, idx1, rows0, rows1, sem0, sem1):
        wid = lax.axis_index("s") * 2 + lax.axis_index("c")
        w0 = wid * (n_chunks * _CH)
        idx_v = (idx0, idx1)
        rows_v = (rows0, rows1)
        sems = (sem0, sem1)
        pltpu.sync_copy(idx_hbm.at[pl.ds(w0, _CH)], idx0)
        cp = pltpu.async_copy(table_hbm.at[idx0], rows0, sem0)
        for c in range(n_chunks):
            p, q = c % 2, (c + 1) % 2
            if c + 1 < n_chunks:
                pltpu.sync_copy(idx_hbm.at[pl.ds(w0 + (c + 1) * _CH, _CH)], idx_v[q])
                ncp = pltpu.async_copy(table_hbm.at[idx_v[q]], rows_v[q], sems[q])
            cp.wait()
            pltpu.sync_copy(rows_v[p], out_hbm.at[pl.ds(w0 + c * _CH, _CH)])
            if c + 1 < n_chunks:
                cp = ncp

    return gk(table, idx)


def _bary_knn(bary):
    baryp = jnp.concatenate(
        [bary, jnp.full((T_PAD - T_REAL, 3), 1e20, jnp.float32)], axis=0)
    baryp = jnp.pad(baryp, ((0, 0), (0, 5)))
    baryT = baryp.T
    nbr = pl.pallas_call(
        _bary_knn_body,
        grid=(T_PAD // QB,),
        in_specs=[
            pl.BlockSpec((QB, 8), lambda i: (i, 0)),
            pl.BlockSpec((8, T_PAD), lambda i: (0, 0)),
        ],
        out_specs=pl.BlockSpec((QB, K_KNN), lambda i: (i, 0)),
        out_shape=jax.ShapeDtypeStruct((T_PAD, K_KNN), jnp.int32),
        compiler_params=pltpu.CompilerParams(dimension_semantics=("parallel",)),
    )(baryp, baryT)
    return nbr[:T_REAL]


def kernel(user_number_triangles, graph_nodes, graph_adjacency_matrix, W1, b1, W2, Wdev, Wq, Wk, Wm1, bm1, Wm2, bm2):
    A = graph_adjacency_matrix
    A_norm = A / (jnp.sum(A, axis=1, keepdims=True) + 1e-6)
    h = jax.nn.relu(A_norm @ (graph_nodes @ W1) + b1)
    inclusion_score = (A_norm @ (h @ W2))[:, 0]
    N_TRI = 500
    target_p = min(graph_nodes.shape[0], N_TRI * 3)
    u = jax.random.uniform(jax.random.key(42), inclusion_score.shape, dtype=jnp.float32)
    g = -jnp.log(-jnp.log(u + 1e-20) + 1e-20)
    _, sel = jax.lax.top_k(jax.lax.stop_gradient(inclusion_score) + g, target_p)
    x = graph_nodes[sel]
    x_sg = x
    d2 = jnp.sum((x_sg[:, None, :] - x_sg[None, :, :]) ** 2, axis=-1)
    _, nn_idx = jax.lax.top_k(-d2, K_SIMPLE + 1)
    knn = nn_idx[:, 1:]
    xdiff = x[knn] - x[:, None, :]
    edge_feat = jax.nn.relu(xdiff @ Wdev)
    f = jnp.mean(edge_feat, axis=1)
    q = f @ Wq
    kk = f @ Wk
    att = jnp.einsum('pd,pkd->pk', q, kk[knn]) / jnp.sqrt(float(D_H))
    S = jax.nn.sigmoid(att)
    P = x.shape[0]
    rows = jnp.broadcast_to(jnp.arange(P)[:, None], knn.shape)
    A_s = jnp.zeros((P, P), dtype=jnp.float32).at[rows, knn].max(S)
    A_s = jnp.maximum(A_s, A_s.T)
    pa, pb = np.triu_indices(NB_PAIR, 1)
    anchor = jnp.broadcast_to(jnp.arange(P)[:, None], (P, pa.shape[0]))
    tri_ids = jnp.stack([anchor, knn[:, pa], knn[:, pb]], axis=-1).reshape(-1, 3)
    triangles = x[tri_ids]
    i0, i1, i2 = tri_ids[:, 0], tri_ids[:, 1], tri_ids[:, 2]
    p_init = A_s[i0, i1] * A_s[i1, i2] * A_s[i0, i2]
    bary = jnp.mean(triangles, axis=1)
    bary_sg = bary
    T = bary.shape[0]
    CH = 500

    indices_neigh_tri = _bary_knn(bary_sg)
    tp = jnp.concatenate(
        [triangles.reshape(T, 9), p_init[:, None], jnp.zeros((T, 118), jnp.float32)],
        axis=1)
    n_chunks = -(-(T_PAD * K_KNN) // (_NW * _CH))
    b_pad = _NW * n_chunks * _CH
    idx_flat = jnp.pad(
        jnp.pad(indices_neigh_tri, ((0, T_PAD - T), (0, 0))).reshape(-1),
        (0, b_pad - T_PAD * K_KNN))
    rowsg = _sc_gather_rows(tp, idx_flat.astype(jnp.int32), n_chunks)
    g16 = rowsg[:T * K_KNN].reshape(T, K_KNN, 128)
    r_matrix = g16[:, :, :9] - jnp.tile(bary, (1, 3))[:, None, :]
    w = g16[:, :, 9:10]
    hm = jax.nn.relu(r_matrix @ Wm1 + bm1)
    pooled = jnp.mean(hm * w, axis=1)
    final_scores = (pooled @ Wm2 + bm2)[:, 0]
    final_scores = final_scores + 0.0 * jnp.asarray(user_number_triangles, dtype=jnp.float32)
    fs_pad = jnp.pad(final_scores[:, None], ((0, T_PAD - T), (0, 0)),
                     constant_values=-jnp.inf)
    tri16 = jnp.pad(triangles.reshape(T, 9), ((0, T_PAD - T), (0, 7)))
    out = _select_top500(fs_pad, tri16)
    return out[:N_TRI, :9].reshape(N_TRI, 3, 3)


# speedup vs baseline: 3.6620x; 1.0006x over previous
import functools

import jax
import jax.numpy as jnp
import numpy as np
from jax import lax
from jax.experimental import pallas as pl
from jax.experimental.pallas import tpu as pltpu
from jax.experimental.pallas import tpu_sc as plsc

N_NODES = 4096
D_H = 64
K_SIMPLE = 15
K_KNN = 20
NB_PAIR = 5

T_REAL = 15000
T_PAD = 15104
QB = 256


def _bary_knn_body(q_ref, ct_ref, o_ref):
    _BIG = jnp.int32(2**30)
    _INF = jnp.float32(jnp.inf)
    q = q_ref[...]
    acc = None
    for d in range(3):
        diff = q[:, d:d + 1] - ct_ref[d:d + 1, :]
        sq = diff * diff
        acc = sq if acc is None else acc + sq
    iota = jax.lax.broadcasted_iota(jnp.int32, (QB, T_PAD), 1)
    d2 = acc
    cols = []
    for k in range(K_KNN):
        m = jnp.min(d2, axis=1, keepdims=True)
        im = jnp.min(jnp.where(d2 == m, iota, _BIG), axis=1, keepdims=True)
        cols.append(im)
        d2 = jnp.where(iota == im, _INF, d2)
    o_ref[...] = jnp.concatenate(cols, axis=1)


RB = 128


def _rank_body(s_ref, st_ref, o_ref):
    s = s_ref[...]
    st = st_ref[...][0:1, :]
    j = jax.lax.broadcasted_iota(jnp.int32, (RB, T_PAD), 1)
    i = (jax.lax.broadcasted_iota(jnp.int32, (RB, 1), 0)
         + pl.program_id(0) * RB)
    gt = (st > s) | ((st == s) & (j < i))
    o_ref[...] = jnp.sum(gt.astype(jnp.int32), axis=1, keepdims=True)


OB = 256


def _build_body(rk_ref, tri_ref, o_ref):
    rk = rk_ref[...][0:1, :]
    r = (jax.lax.broadcasted_iota(jnp.int32, (OB, 1), 0)
         + pl.program_id(0) * OB)
    M = (rk == r).astype(jnp.float32)
    o_ref[...] = jnp.dot(M, tri_ref[...], preferred_element_type=jnp.float32, precision=jax.lax.Precision.HIGHEST)


def _select_top500(fs_pad, tri16):
    rank = pl.pallas_call(
        _rank_body,
        grid=(T_PAD // RB,),
        in_specs=[
            pl.BlockSpec((RB, 1), lambda i: (i, 0)),
            pl.BlockSpec((8, T_PAD), lambda i: (0, 0)),
        ],
        out_specs=pl.BlockSpec((RB, 1), lambda i: (i, 0)),
        out_shape=jax.ShapeDtypeStruct((T_PAD, 1), jnp.int32),
    )(fs_pad, jnp.broadcast_to(fs_pad.T, (8, T_PAD)))
    out = pl.pallas_call(
        _build_body,
        grid=(2,),
        in_specs=[
            pl.BlockSpec((8, T_PAD), lambda i: (0, 0)),
            pl.BlockSpec((T_PAD, 16), lambda i: (0, 0)),
        ],
        out_specs=pl.BlockSpec((OB, 16), lambda i: (i, 0)),
        out_shape=jax.ShapeDtypeStruct((2 * OB, 16), jnp.float32),
    )(jnp.broadcast_to(rank.T, (8, T_PAD)), tri16)
    return out


_NW = 32
_CH = 384


def _sc_gather_rows(table, idx, n_chunks):
    mesh = plsc.VectorSubcoreMesh(core_axis_name="c", subcore_axis_name="s")
    b_total = idx.shape[0]

    @functools.partial(
        pl.kernel,
        mesh=mesh,
        out_type=jax.ShapeDtypeStruct((b_total, 128), jnp.float32),
        scratch_types=[
            pltpu.VMEM((_CH,), jnp.int32),
            pltpu.VMEM((_CH,), jnp.int32),
            pltpu.VMEM((_CH, 128), jnp.float32),
            pltpu.VMEM((_CH, 128), jnp.float32),
            pltpu.SemaphoreType.DMA,
            pltpu.SemaphoreType.DMA,
        ],
    )
    def gk(table_hbm, idx_hbm, out_hbm, idx0, idx1, rows0, rows1, sem0, sem1):
        wid = lax.axis_index("s") * 2 + lax.axis_index("c")
        w0 = wid * (n_chunks * _CH)
        idx_v = (idx0, idx1)
        rows_v = (rows0, rows1)
        sems = (sem0, sem1)
        pltpu.sync_copy(idx_hbm.at[pl.ds(w0, _CH)], idx0)
        cp = pltpu.async_copy(table_hbm.at[idx0], rows0, sem0)
        for c in range(n_chunks):
            p, q = c % 2, (c + 1) % 2
            if c + 1 < n_chunks:
                pltpu.sync_copy(idx_hbm.at[pl.ds(w0 + (c + 1) * _CH, _CH)], idx_v[q])
                ncp = pltpu.async_copy(table_hbm.at[idx_v[q]], rows_v[q], sems[q])
            cp.wait()
            pltpu.sync_copy(rows_v[p], out_hbm.at[pl.ds(w0 + c * _CH, _CH)])
            if c + 1 < n_chunks:
                cp = ncp

    return gk(table, idx)


def _bary_knn(bary):
    baryp = jnp.concatenate(
        [bary, jnp.full((T_PAD - T_REAL, 3), 1e20, jnp.float32)], axis=0)
    baryp = jnp.pad(baryp, ((0, 0), (0, 5)))
    baryT = baryp.T
    nbr = pl.pallas_call(
        _bary_knn_body,
        grid=(T_PAD // QB,),
        in_specs=[
            pl.BlockSpec((QB, 8), lambda i: (i, 0)),
            pl.BlockSpec((8, T_PAD), lambda i: (0, 0)),
        ],
        out_specs=pl.BlockSpec((QB, K_KNN), lambda i: (i, 0)),
        out_shape=jax.ShapeDtypeStruct((T_PAD, K_KNN), jnp.int32),
        compiler_params=pltpu.CompilerParams(dimension_semantics=("parallel",)),
    )(baryp, baryT)
    return nbr[:T_REAL]


def kernel(user_number_triangles, graph_nodes, graph_adjacency_matrix, W1, b1, W2, Wdev, Wq, Wk, Wm1, bm1, Wm2, bm2):
    A = graph_adjacency_matrix
    A_norm = A / (jnp.sum(A, axis=1, keepdims=True) + 1e-6)
    h = jax.nn.relu(A_norm @ (graph_nodes @ W1) + b1)
    inclusion_score = (A_norm @ (h @ W2))[:, 0]
    N_TRI = 500
    target_p = min(graph_nodes.shape[0], N_TRI * 3)
    u = jax.random.uniform(jax.random.key(42), inclusion_score.shape, dtype=jnp.float32)
    g = -jnp.log(-jnp.log(u + 1e-20) + 1e-20)
    _, sel = jax.lax.top_k(jax.lax.stop_gradient(inclusion_score) + g, target_p)
    x = graph_nodes[sel]
    x_sg = x
    d2 = jnp.sum((x_sg[:, None, :] - x_sg[None, :, :]) ** 2, axis=-1)
    _, nn_idx = jax.lax.top_k(-d2, K_SIMPLE + 1)
    knn = nn_idx[:, 1:]
    xdiff = x[knn] - x[:, None, :]
    edge_feat = jax.nn.relu(xdiff @ Wdev)
    f = jnp.mean(edge_feat, axis=1)
    q = f @ Wq
    kk = f @ Wk
    att = jnp.einsum('pd,pkd->pk', q, kk[knn]) / jnp.sqrt(float(D_H))
    S = jax.nn.sigmoid(att)
    P = x.shape[0]
    rows = jnp.broadcast_to(jnp.arange(P)[:, None], knn.shape)
    A_s = jnp.zeros((P, P), dtype=jnp.float32).at[rows, knn].max(S)
    A_s = jnp.maximum(A_s, A_s.T)
    pa, pb = np.triu_indices(NB_PAIR, 1)
    anchor = jnp.broadcast_to(jnp.arange(P)[:, None], (P, pa.shape[0]))
    tri_ids = jnp.stack([anchor, knn[:, pa], knn[:, pb]], axis=-1).reshape(-1, 3)
    triangles = x[tri_ids]
    i0, i1, i2 = tri_ids[:, 0], tri_ids[:, 1], tri_ids[:, 2]
    p_init = A_s[i0, i1] * A_s[i1, i2] * A_s[i0, i2]
    bary = jnp.mean(triangles, axis=1)
    bary_sg = bary
    T = bary.shape[0]

    indices_neigh_tri = _bary_knn(bary_sg)
    tp = jnp.concatenate(
        [triangles.reshape(T, 9), p_init[:, None], jnp.zeros((T, 118), jnp.float32)],
        axis=1)
    n_chunks = -(-(T_PAD * K_KNN) // (_NW * _CH))
    b_pad = _NW * n_chunks * _CH
    idx_flat = jnp.pad(
        jnp.pad(indices_neigh_tri, ((0, T_PAD - T), (0, 0))).reshape(-1),
        (0, b_pad - T_PAD * K_KNN))
    rowsg = _sc_gather_rows(tp, idx_flat.astype(jnp.int32), n_chunks)
    g16 = rowsg[:T * K_KNN].reshape(T, K_KNN, 128)
    r_matrix = g16[:, :, :9] - jnp.tile(bary, (1, 3))[:, None, :]
    w = g16[:, :, 9:10]
    hm = jax.nn.relu(r_matrix @ Wm1 + bm1)
    pooled = jnp.mean(hm * w, axis=1)
    final_scores = (pooled @ Wm2 + bm2)[:, 0]
    final_scores = final_scores + 0.0 * jnp.asarray(user_number_triangles, dtype=jnp.float32)
    fs_pad = jnp.pad(final_scores[:, None], ((0, T_PAD - T), (0, 0)),
                     constant_values=-jnp.inf)
    tri16 = jnp.pad(triangles.reshape(T, 9), ((0, T_PAD - T), (0, 7)))
    out = _select_top500(fs_pad, tri16)
    return out[:N_TRI, :9].reshape(N_TRI, 3, 3)
